# baseline (speedup 1.0000x reference)
import functools

import jax
import jax.numpy as jnp
from jax import lax
from jax.experimental import pallas as pl
from jax.experimental.pallas import tpu as pltpu
from jax.experimental.pallas import tpu_sc as plsc

F = 26
B = 16384
VOCAB = 100000
V = F * VOCAB
VMAIN = 2599936
NC = 2
NS = 16
L = 16
NW = NC * NS
BPW = B // NW
CPW = BPW * F

NCHUNK = 8
CHUNK = CPW // NCHUNK
SLICES = CPW // L
SPC = SLICES // NCHUNK
RPF = BPW // L


def _body(x_hbm, wm_hbm, wt_hbm, b_hbm, out_hbm,
          x_v, idc_v, g_v, acc_v, tail_v, bias_v, sem):
    wid = lax.axis_index("s") * NC + lax.axis_index("c")

    pltpu.sync_copy(x_hbm.at[pl.ds(wid * BPW, BPW), :], x_v)
    pltpu.sync_copy(wt_hbm, tail_v)
    pltpu.sync_copy(b_hbm, bias_v)

    iota = lax.iota(jnp.int32, L)

    copies = []
    for c in range(NCHUNK):

        def sl_body(i, carry, c=c):
            for u in range(4):
                s = c * SPC + i * 4 + u
                f = s >> 5
                r0 = (s & 31) * L
                raw = plsc.load_gather(
                    x_v, [r0 + iota, f + jnp.zeros((L,), jnp.int32)]
                )
                idc_v[pl.ds(s * L, L)] = jnp.minimum(raw + f * VOCAB, VMAIN - 1)
            return carry

        lax.fori_loop(0, SPC // 4, sl_body, 0)
        copies.append(
            pltpu.async_copy(
                wm_hbm.at[idc_v.at[pl.ds(c * CHUNK, CHUNK)]],
                g_v.at[pl.ds(c * CHUNK, CHUNK)],
                sem,
            )
        )
    for cp in copies:
        cp.wait()

    def fix_body(t, carry):
        r0 = t * L
        raw = plsc.load_gather(
            x_v, [r0 + iota, (F - 1) + jnp.zeros((L,), jnp.int32)]
        )
        iv = raw + (F - 1) * VOCAB
        m = iv >= VMAIN
        tpos = jnp.clip(iv - VMAIN, 0, V - VMAIN - 1)
        tv = plsc.load_gather(tail_v, [tpos])
        j = (F - 1) * BPW + r0
        g_v[pl.ds(j, L)] = jnp.where(m, tv, g_v[pl.ds(j, L)])
        return carry

    lax.fori_loop(0, RPF, fix_body, 0)

    bias_vec = bias_v[...]

    def red_body(t, carry):
        r0 = t * L
        s = g_v[pl.ds(r0, L)]
        for f in range(1, F):
            s = s + g_v[pl.ds(f * BPW + r0, L)]
        acc_v[pl.ds(r0, L)] = s / float(F) + bias_vec
        return carry

    lax.fori_loop(0, RPF, red_body, 0)

    pltpu.sync_copy(acc_v, out_hbm.at[pl.ds(wid * BPW, BPW)])


@jax.jit
def _emb(xp, wm, wt, b16):
    mesh = plsc.VectorSubcoreMesh(core_axis_name="c", subcore_axis_name="s")
    run = functools.partial(
        pl.kernel,
        mesh=mesh,
        out_type=jax.ShapeDtypeStruct((B,), jnp.float32),
        scratch_types=[
            pltpu.VMEM((BPW, F), jnp.int32),
            pltpu.VMEM((CPW,), jnp.int32),
            pltpu.VMEM((CPW,), jnp.float32),
            pltpu.VMEM((BPW,), jnp.float32),
            pltpu.VMEM((V - VMAIN,), jnp.float32),
            pltpu.VMEM((L,), jnp.float32),
            pltpu.SemaphoreType.DMA,
        ],
        compiler_params=pltpu.CompilerParams(needs_layout_passes=False),
    )(_body)
    return run(xp, wm, wt, b16)


def kernel(x, weight, bias):
    xp = x.astype(jnp.int32)
    b16 = jnp.broadcast_to(bias.astype(jnp.float32), (L,))
    w_main = weight[:VMAIN].reshape(-1)
    w_tail = weight[VMAIN:].reshape(-1)
    out = _emb(xp, w_main, w_tail, b16)
    return out.reshape(B, 1)


# speedup vs baseline: 1.0196x; 1.0196x over previous
import functools

import jax
import jax.numpy as jnp
from jax import lax
from jax.experimental import pallas as pl
from jax.experimental.pallas import tpu as pltpu
from jax.experimental.pallas import tpu_sc as plsc

F = 26
B = 16384
VOCAB = 100000
V = F * VOCAB
VMAIN = 2599936
NC = 2
NS = 16
L = 16
NW = NC * NS
BPW = B // NW
CPW = BPW * F

NCHUNK = 16
CHUNK = CPW // NCHUNK
SLICES = CPW // L
SPC = SLICES // NCHUNK
RPF = BPW // L


def _body(x_hbm, wm_hbm, wt_hbm, b_hbm, out_hbm,
          x_v, idc_v, g_v, acc_v, tail_v, bias_v, sem):
    wid = lax.axis_index("s") * NC + lax.axis_index("c")

    pltpu.sync_copy(x_hbm.at[pl.ds(wid * BPW, BPW), :], x_v)
    pltpu.sync_copy(wt_hbm, tail_v)
    pltpu.sync_copy(b_hbm, bias_v)

    iota = lax.iota(jnp.int32, L)

    copies = []
    for c in range(NCHUNK):

        def sl_body(i, carry, c=c):
            s = c * SPC + i
            f = s >> 5
            r0 = (s & 31) * L
            raw = plsc.load_gather(x_v, [r0 + iota, f + jnp.zeros((L,), jnp.int32)])
            idc_v[pl.ds(s * L, L)] = jnp.minimum(raw + f * VOCAB, VMAIN - 1)
            return carry

        lax.fori_loop(0, SPC, sl_body, 0)
        copies.append(
            pltpu.async_copy(
                wm_hbm.at[idc_v.at[pl.ds(c * CHUNK, CHUNK)]],
                g_v.at[pl.ds(c * CHUNK, CHUNK)],
                sem,
            )
        )
    for cp in copies:
        cp.wait()

    def fix_body(t, carry):
        r0 = t * L
        raw = plsc.load_gather(
            x_v, [r0 + iota, (F - 1) + jnp.zeros((L,), jnp.int32)]
        )
        iv = raw + (F - 1) * VOCAB
        m = iv >= VMAIN
        tpos = jnp.clip(iv - VMAIN, 0, V - VMAIN - 1)
        tv = plsc.load_gather(tail_v, [tpos])
        j = (F - 1) * BPW + r0
        g_v[pl.ds(j, L)] = jnp.where(m, tv, g_v[pl.ds(j, L)])
        return carry

    lax.fori_loop(0, RPF, fix_body, 0)

    bias_vec = bias_v[...]

    def red_body(t, carry):
        r0 = t * L
        s = g_v[pl.ds(r0, L)]
        for f in range(1, F):
            s = s + g_v[pl.ds(f * BPW + r0, L)]
        acc_v[pl.ds(r0, L)] = s / float(F) + bias_vec
        return carry

    lax.fori_loop(0, RPF, red_body, 0)

    pltpu.sync_copy(acc_v, out_hbm.at[pl.ds(wid * BPW, BPW)])


@jax.jit
def _emb(xp, wm, wt, b16):
    mesh = plsc.VectorSubcoreMesh(core_axis_name="c", subcore_axis_name="s")
    run = functools.partial(
        pl.kernel,
        mesh=mesh,
        out_type=jax.ShapeDtypeStruct((B,), jnp.float32),
        scratch_types=[
            pltpu.VMEM((BPW, F), jnp.int32),
            pltpu.VMEM((CPW,), jnp.int32),
            pltpu.VMEM((CPW,), jnp.float32),
            pltpu.VMEM((BPW,), jnp.float32),
            pltpu.VMEM((V - VMAIN,), jnp.float32),
            pltpu.VMEM((L,), jnp.float32),
            pltpu.SemaphoreType.DMA,
        ],
        compiler_params=pltpu.CompilerParams(needs_layout_passes=False),
    )(_body)
    return run(xp, wm, wt, b16)


def kernel(x, weight, bias):
    xp = x.astype(jnp.int32)
    b16 = jnp.broadcast_to(bias.astype(jnp.float32), (L,))
    w_main = weight[:VMAIN].reshape(-1)
    w_tail = weight[VMAIN:].reshape(-1)
    out = _emb(xp, w_main, w_tail, b16)
    return out.reshape(B, 1)
